# Initial kernel scaffold; baseline (speedup 1.0000x reference)
import jax
import jax.numpy as jnp
from jax.experimental import pallas as pl


def kernel(xs, W_body, b_body, embed_weight):
    raise NotImplementedError("write your pallas kernel here")


# speedup vs baseline: 1.2030x; 1.2030x over previous
import functools

import jax
import jax.numpy as jnp
from jax import lax
from jax.experimental import pallas as pl
from jax.experimental.pallas import tpu as pltpu
from jax.experimental.pallas import tpu_sc as plsc

N_BLOCK = 256


def _argmin_body(x_ref, w_ref, b_ref, et_ref, idx_ref, loss_ref, *, scale):
    i = pl.program_id(0)
    feat = jnp.dot(x_ref[...], w_ref[...],
                   preferred_element_type=jnp.float32) + b_ref[...]
    mm = lax.dot_general(feat, et_ref[...], (((1,), (0,)), ((), ())),
                         preferred_element_type=jnp.float32)
    sq = feat * feat
    t = sq[:, 0:8]
    for g in range(1, sq.shape[1] // 8):
        t = t + sq[:, 8 * g: 8 * g + 8]
    while t.shape[1] > 1:
        h = t.shape[1] // 2
        t = t[:, :h] + t[:, h:]
    feat_sq = t
    e_sq = jnp.sum(et_ref[...] * et_ref[...], axis=0, keepdims=True)
    d = (feat_sq + e_sq) - 2.0 * mm
    min_d = jnp.min(d, axis=1, keepdims=True)
    k = d.shape[1]
    iota = lax.broadcasted_iota(jnp.int32, d.shape, 1)
    idx = jnp.min(jnp.where(d == min_d, iota, k), axis=1)
    idx_ref[0, 0, :] = idx
    part = jnp.reshape(jnp.sum(min_d) * scale, (1, 1))

    @pl.when(i == 0)
    def _init():
        loss_ref[...] = part

    @pl.when(i > 0)
    def _acc():
        loss_ref[...] += part


def _compute_indices_and_loss(xs, w, b2, e_t):
    n, d_in = xs.shape
    d, k = e_t.shape
    nb = n // N_BLOCK
    scale = 1.25 / (n * d)
    idx3, loss2 = pl.pallas_call(
        functools.partial(_argmin_body, scale=scale),
        grid=(nb,),
        in_specs=[
            pl.BlockSpec((N_BLOCK, d_in), lambda i: (i, 0)),
            pl.BlockSpec((d_in, d), lambda i: (0, 0)),
            pl.BlockSpec((1, d), lambda i: (0, 0)),
            pl.BlockSpec((d, k), lambda i: (0, 0)),
        ],
        out_specs=[
            pl.BlockSpec((1, 1, N_BLOCK), lambda i: (i, 0, 0)),
            pl.BlockSpec((1, 1), lambda i: (0, 0)),
        ],
        out_shape=[
            jax.ShapeDtypeStruct((nb, 1, N_BLOCK), jnp.int32),
            jax.ShapeDtypeStruct((1, 1), jnp.float32),
        ],
    )(xs, w, b2, e_t)
    return idx3.reshape(n), loss2[0, 0]


def _sc_gather(table, idx):
    n = idx.shape[0]
    _, d = table.shape
    info = plsc.get_sparse_core_info()
    nw = info.num_cores * info.num_subcores
    b_per_w = n // nw
    chunk = 128
    n_chunks = b_per_w // chunk
    idx2 = idx.reshape(n // chunk, chunk)
    mesh = plsc.VectorSubcoreMesh(core_axis_name="c", subcore_axis_name="s")

    @functools.partial(
        pl.kernel, mesh=mesh,
        compiler_params=pltpu.CompilerParams(use_tc_tiling_on_sc=False),
        out_type=jax.ShapeDtypeStruct((n, d), jnp.float32),
        scratch_types=[
            pltpu.VMEM((n_chunks, chunk), jnp.int32),
            pltpu.VMEM((b_per_w, d), jnp.float32),
            pltpu.SemaphoreType.DMA,
        ],
    )
    def gather_kernel(table_hbm, idx_hbm, out_hbm, idx_v, rows_v, sem):
        wid = lax.axis_index("s") * info.num_cores + lax.axis_index("c")
        base = wid * b_per_w
        pltpu.sync_copy(idx_hbm.at[pl.ds(wid * n_chunks, n_chunks)], idx_v)
        copies = []
        for j in range(n_chunks):
            copies.append(pltpu.async_copy(
                table_hbm.at[idx_v.at[j]],
                rows_v.at[pl.ds(j * chunk, chunk)], sem))
        for c in copies:
            c.wait()
        pltpu.sync_copy(rows_v, out_hbm.at[pl.ds(base, b_per_w)])

    return gather_kernel(table, idx2)


def kernel(xs, W_body, b_body, embed_weight):
    n, d_in = xs.shape
    k, d = embed_weight.shape
    b2 = b_body.reshape(1, d)
    e_t = embed_weight.T
    idx, loss = _compute_indices_and_loss(xs, W_body, b2, e_t)
    output_x = _sc_gather(embed_weight, idx)
    return output_x, loss
